# Initial kernel scaffold; baseline (speedup 1.0000x reference)
import jax
import jax.numpy as jnp
from jax.experimental import pallas as pl


def kernel(text, adj, weight, bias):
    raise NotImplementedError("write your pallas kernel here")


# speedup vs baseline: 4.2411x; 4.2411x over previous
import functools

import jax
import jax.numpy as jnp
from jax.experimental import pallas as pl
from jax.experimental.pallas import tpu as pltpu


def _round_up(x: int, m: int) -> int:
    return ((x + m - 1) // m) * m


_BB = 2


def _fused_gcn_kernel(text_ref, adj_ref, w_ref, b_ref, out_ref, *, bb, n):
    f_in = w_ref.shape[0]
    x = text_ref[...].reshape(bb * n, f_in).astype(jnp.bfloat16)
    h = jnp.dot(x, w_ref[...], preferred_element_type=jnp.float32)
    h = (h + b_ref[...]).astype(jnp.bfloat16)
    for i in range(bb):
        adj = adj_ref[i]
        agg = jnp.dot(adj.astype(jnp.bfloat16), h[i * n:(i + 1) * n],
                      preferred_element_type=jnp.float32)
        denom = jnp.sum(adj, axis=1, keepdims=True) + 1.0
        inv = pl.reciprocal(denom, approx=False)
        out_ref[i] = (agg * inv).astype(out_ref.dtype)


def kernel(text, adj, weight, bias):
    B, N, F_in = text.shape
    F_out = weight.shape[0]

    N_pad = _round_up(N, 128)
    F_in_pad = _round_up(F_in, 128)
    F_out_pad = _round_up(F_out, 128)
    bb = _BB if B % _BB == 0 else 1
    B_pad = _round_up(B, bb)

    f32 = jnp.float32
    text_p = jnp.pad(text.astype(f32),
                     ((0, B_pad - B), (0, N_pad - N), (0, F_in_pad - F_in)))
    adj_p = jnp.pad(adj.astype(f32),
                    ((0, B_pad - B), (0, N_pad - N), (0, N_pad - N)))
    w_p = jnp.zeros((F_in_pad, F_out_pad), jnp.bfloat16)
    w_p = w_p.at[:F_in, :F_out].set(weight.astype(jnp.bfloat16).T)
    b_p = jnp.zeros((1, F_out_pad), f32).at[0, :F_out].set(bias.astype(f32))

    body = functools.partial(_fused_gcn_kernel, bb=bb, n=N_pad)
    out_p = pl.pallas_call(
        body,
        out_shape=jax.ShapeDtypeStruct((B_pad, N_pad, F_out_pad), text.dtype),
        grid=(B_pad // bb,),
        in_specs=[
            pl.BlockSpec((bb, N_pad, F_in_pad), lambda i: (i, 0, 0)),
            pl.BlockSpec((bb, N_pad, N_pad), lambda i: (i, 0, 0)),
            pl.BlockSpec((F_in_pad, F_out_pad), lambda i: (0, 0)),
            pl.BlockSpec((1, F_out_pad), lambda i: (0, 0)),
        ],
        out_specs=pl.BlockSpec((bb, N_pad, F_out_pad), lambda i: (i, 0, 0)),
        compiler_params=pltpu.CompilerParams(
            dimension_semantics=("parallel",)),
    )(text_p, adj_p, w_p, b_p)

    return out_p[:B, :N, :F_out]


# speedup vs baseline: 5.1824x; 1.2220x over previous
import functools

import jax
import jax.numpy as jnp
from jax.experimental import pallas as pl
from jax.experimental.pallas import tpu as pltpu


def _round_up(x: int, m: int) -> int:
    return ((x + m - 1) // m) * m


_BB = 4


def _fused_gcn_kernel(text_ref, adj_ref, w_ref, b_ref, out_ref, *, bb, n):
    f_in = w_ref.shape[0]
    x = text_ref[...].reshape(bb * n, f_in).astype(jnp.bfloat16)
    h = jnp.dot(x, w_ref[...], preferred_element_type=jnp.float32)
    h = (h + b_ref[...]).astype(jnp.bfloat16)
    for i in range(bb):
        adj = adj_ref[i]
        agg = jnp.dot(adj.astype(jnp.bfloat16), h[i * n:(i + 1) * n],
                      preferred_element_type=jnp.float32)
        denom = jnp.sum(adj, axis=1, keepdims=True) + 1.0
        inv = pl.reciprocal(denom, approx=False)
        out_ref[i] = (agg * inv).astype(out_ref.dtype)


def kernel(text, adj, weight, bias):
    B, N, F_in = text.shape
    F_out = weight.shape[0]

    N_pad = _round_up(N, 128)
    F_in_pad = _round_up(F_in, 128)
    F_out_pad = _round_up(F_out, 128)
    bb = _BB if B % _BB == 0 else 1
    B_pad = _round_up(B, bb)

    f32 = jnp.float32
    text_p = jnp.pad(text.astype(f32),
                     ((0, B_pad - B), (0, N_pad - N), (0, F_in_pad - F_in)))
    adj_p = jnp.pad(adj.astype(f32),
                    ((0, B_pad - B), (0, N_pad - N), (0, N_pad - N)))
    w_p = jnp.zeros((F_in_pad, F_out_pad), jnp.bfloat16)
    w_p = w_p.at[:F_in, :F_out].set(weight.astype(jnp.bfloat16).T)
    b_p = jnp.zeros((1, F_out_pad), f32).at[0, :F_out].set(bias.astype(f32))

    body = functools.partial(_fused_gcn_kernel, bb=bb, n=N_pad)
    out_p = pl.pallas_call(
        body,
        out_shape=jax.ShapeDtypeStruct((B_pad, N_pad, F_out_pad), text.dtype),
        grid=(B_pad // bb,),
        in_specs=[
            pl.BlockSpec((bb, N_pad, F_in_pad), lambda i: (i, 0, 0)),
            pl.BlockSpec((bb, N_pad, N_pad), lambda i: (i, 0, 0)),
            pl.BlockSpec((F_in_pad, F_out_pad), lambda i: (0, 0)),
            pl.BlockSpec((1, F_out_pad), lambda i: (0, 0)),
        ],
        out_specs=pl.BlockSpec((bb, N_pad, F_out_pad), lambda i: (i, 0, 0)),
        compiler_params=pltpu.CompilerParams(
            dimension_semantics=("parallel",)),
    )(text_p, adj_p, w_p, b_p)

    return out_p[:B, :N, :F_out]


# speedup vs baseline: 5.3657x; 1.0354x over previous
import functools

import jax
import jax.numpy as jnp
from jax.experimental import pallas as pl
from jax.experimental.pallas import tpu as pltpu


def _round_up(x: int, m: int) -> int:
    return ((x + m - 1) // m) * m


_BB = 8


def _fused_gcn_kernel(text_ref, adj_ref, w_ref, b_ref, out_ref, *, bb, n):
    f_in = w_ref.shape[0]
    x = text_ref[...].reshape(bb * n, f_in).astype(jnp.bfloat16)
    h = jnp.dot(x, w_ref[...], preferred_element_type=jnp.float32)
    h = (h + b_ref[...]).astype(jnp.bfloat16)
    for i in range(bb):
        adj = adj_ref[i]
        agg = jnp.dot(adj.astype(jnp.bfloat16), h[i * n:(i + 1) * n],
                      preferred_element_type=jnp.float32)
        denom = jnp.sum(adj, axis=1, keepdims=True) + 1.0
        inv = pl.reciprocal(denom, approx=False)
        out_ref[i] = (agg * inv).astype(out_ref.dtype)


def kernel(text, adj, weight, bias):
    B, N, F_in = text.shape
    F_out = weight.shape[0]

    N_pad = _round_up(N, 128)
    F_in_pad = _round_up(F_in, 128)
    F_out_pad = _round_up(F_out, 128)
    bb = _BB if B % _BB == 0 else 1
    B_pad = _round_up(B, bb)

    f32 = jnp.float32
    text_p = jnp.pad(text.astype(f32),
                     ((0, B_pad - B), (0, N_pad - N), (0, F_in_pad - F_in)))
    adj_p = jnp.pad(adj.astype(f32),
                    ((0, B_pad - B), (0, N_pad - N), (0, N_pad - N)))
    w_p = jnp.zeros((F_in_pad, F_out_pad), jnp.bfloat16)
    w_p = w_p.at[:F_in, :F_out].set(weight.astype(jnp.bfloat16).T)
    b_p = jnp.zeros((1, F_out_pad), f32).at[0, :F_out].set(bias.astype(f32))

    body = functools.partial(_fused_gcn_kernel, bb=bb, n=N_pad)
    out_p = pl.pallas_call(
        body,
        out_shape=jax.ShapeDtypeStruct((B_pad, N_pad, F_out_pad), text.dtype),
        grid=(B_pad // bb,),
        in_specs=[
            pl.BlockSpec((bb, N_pad, F_in_pad), lambda i: (i, 0, 0)),
            pl.BlockSpec((bb, N_pad, N_pad), lambda i: (i, 0, 0)),
            pl.BlockSpec((F_in_pad, F_out_pad), lambda i: (0, 0)),
            pl.BlockSpec((1, F_out_pad), lambda i: (0, 0)),
        ],
        out_specs=pl.BlockSpec((bb, N_pad, F_out_pad), lambda i: (i, 0, 0)),
        compiler_params=pltpu.CompilerParams(
            dimension_semantics=("parallel",)),
    )(text_p, adj_p, w_p, b_p)

    return out_p[:B, :N, :F_out]
